# baseline (speedup 1.0000x reference)
import functools

import jax
import jax.numpy as jnp
from jax import lax
from jax.experimental import pallas as pl
from jax.experimental.pallas import tpu as pltpu
from jax.experimental.pallas import tpu_sc as plsc

VOCAB = 100000
EMBED_DIM = 128
BATCH = 4096
HIST_LEN = 50

NUM_CORES = 2
NUM_SUBCORES = 16
CHUNK = 128
SC_COLS = NUM_SUBCORES * CHUNK
TOTAL = BATCH * HIST_LEN


def _make_kernel():
    mesh = plsc.VectorSubcoreMesh(core_axis_name="c", subcore_axis_name="s")

    @functools.partial(
        pl.kernel,
        mesh=mesh,
        out_type=jax.ShapeDtypeStruct((TOTAL, EMBED_DIM), jnp.float32),
        scratch_types=[
            pltpu.VMEM((HIST_LEN, CHUNK), jnp.int32),
            pltpu.VMEM((CHUNK, EMBED_DIM), jnp.float32),
            pltpu.VMEM((CHUNK, EMBED_DIM), jnp.float32),
            pltpu.VMEM_SHARED((2, SC_COLS, EMBED_DIM), jnp.float32),
            pltpu.SemaphoreType.DMA,
            pltpu.SemaphoreType.DMA,
            pltpu.SemaphoreType.DMA,
            pltpu.SemaphoreType.DMA,
        ],
    )
    def gather_kernel(
        idx_hbm, table_hbm, out_hbm, idx_v, buf0, buf1, slab, g0, g1, os0, os1
    ):
        c = lax.axis_index("c")
        s = lax.axis_index("s")
        bufs = (buf0, buf1)
        gsem = (g0, g1)
        osem = (os0, os1)
        col = c * SC_COLS + s * CHUNK
        pltpu.sync_copy(idx_hbm.at[:, pl.ds(col, CHUNK)], idx_v)

        def out_slice(l):
            return out_hbm.at[pl.ds(l * BATCH + c * SC_COLS, SC_COLS)]

        for p in range(2):
            pltpu.async_copy(table_hbm.at[idx_v.at[p]], bufs[p], gsem[p])

        def body(i, carry):
            for p in range(2):
                l = 2 * i + p

                @pl.when((s == 0) & (i > 0))
                def _():
                    pltpu.make_async_copy(
                        slab.at[p], out_slice(l - 2), osem[p]
                    ).wait()

                plsc.subcore_barrier()
                pltpu.make_async_copy(
                    table_hbm.at[idx_v.at[l]], bufs[p], gsem[p]
                ).wait()
                pltpu.sync_copy(bufs[p], slab.at[p, pl.ds(s * CHUNK, CHUNK)])

                @pl.when(l + 2 < HIST_LEN)
                def _():
                    pltpu.async_copy(
                        table_hbm.at[idx_v.at[l + 2]], bufs[p], gsem[p]
                    )

                plsc.subcore_barrier()

                @pl.when(s == 0)
                def _():
                    pltpu.async_copy(slab.at[p], out_slice(l), osem[p])

            return carry

        lax.fori_loop(0, HIST_LEN // 2, body, 0)

        @pl.when(s == 0)
        def _():
            for p in range(2):
                pltpu.make_async_copy(
                    slab.at[p], out_slice(HIST_LEN - 2 + p), osem[p]
                ).wait()

    return gather_kernel


_kernel_fn = _make_kernel()


def kernel(indices, table):
    out = _kernel_fn(indices.T, table)
    return out.reshape(HIST_LEN, BATCH, EMBED_DIM).transpose(1, 0, 2)


# speedup vs baseline: 1.0249x; 1.0249x over previous
import functools

import jax
import jax.numpy as jnp
from jax import lax
from jax.experimental import pallas as pl
from jax.experimental.pallas import tpu as pltpu
from jax.experimental.pallas import tpu_sc as plsc

VOCAB = 100000
EMBED_DIM = 128
BATCH = 4096
HIST_LEN = 50

NUM_CORES = 2
NUM_SUBCORES = 16
NW = NUM_CORES * NUM_SUBCORES
TOTAL = BATCH * HIST_LEN
B_PER_W = TOTAL // NW
CHUNK = 128
N_CHUNKS = B_PER_W // CHUNK
NBUF = 5
N_OUTER = N_CHUNKS // NBUF


def _make_kernel():
    mesh = plsc.VectorSubcoreMesh(core_axis_name="c", subcore_axis_name="s")

    @functools.partial(
        pl.kernel,
        mesh=mesh,
        out_type=jax.ShapeDtypeStruct((TOTAL, EMBED_DIM), jnp.float32),
        scratch_types=[
            pltpu.VMEM((N_CHUNKS, CHUNK), jnp.int32),
        ]
        + [pltpu.VMEM((CHUNK, EMBED_DIM), jnp.float32) for _ in range(NBUF)]
        + [pltpu.SemaphoreType.DMA for _ in range(2 * NBUF)],
    )
    def gather_kernel(idx_hbm, table_hbm, out_hbm, idx_v, *scratch):
        bufs = scratch[:NBUF]
        gsem = scratch[NBUF : 2 * NBUF]
        osem = scratch[2 * NBUF :]
        wid = lax.axis_index("s") * NUM_CORES + lax.axis_index("c")
        col = wid * CHUNK
        pltpu.sync_copy(idx_hbm.at[:, pl.ds(col, CHUNK)], idx_v)

        def out_slice(j):
            return out_hbm.at[pl.ds(j * BATCH + col, CHUNK)]

        for b in range(NBUF):
            pltpu.async_copy(table_hbm.at[idx_v.at[b]], bufs[b], gsem[b])

        def body(i, carry):
            for b in range(NBUF):
                j = i * NBUF + b
                pltpu.make_async_copy(
                    table_hbm.at[idx_v.at[j]], bufs[b], gsem[b]
                ).wait()
                pltpu.async_copy(bufs[b], out_slice(j), osem[b])
            for b in range(NBUF):
                j = i * NBUF + b
                pltpu.make_async_copy(bufs[b], out_slice(j), osem[b]).wait()

                @pl.when(i < N_OUTER - 1)
                def _():
                    pltpu.async_copy(
                        table_hbm.at[idx_v.at[j + NBUF]], bufs[b], gsem[b]
                    )

            return carry

        lax.fori_loop(0, N_OUTER, body, 0)

    return gather_kernel


_kernel_fn = _make_kernel()


def kernel(indices, table):
    out = _kernel_fn(indices.T, table)
    return out.reshape(HIST_LEN, BATCH, EMBED_DIM).transpose(1, 0, 2)
